# Initial kernel scaffold; baseline (speedup 1.0000x reference)
import jax
import jax.numpy as jnp
from jax.experimental import pallas as pl


def kernel(x_word, x_doc, edges_w2w, edges_w2d, edges_w2wr, edges_d2w, Wself1, Wneigh1, b1, Wself2, Wneigh2, b2, lin_w, lin_b):
    raise NotImplementedError("write your pallas kernel here")


# speedup vs baseline: 1.0943x; 1.0943x over previous
import jax
import jax.numpy as jnp
from jax.experimental import pallas as pl

N_WORD = 10000
N_DOC = 10000


def _sage(x_src, x_dst, edges, Wself, Wneigh, b, n_dst):
    src, dst = edges[0], edges[1]
    z = x_src @ Wneigh
    agg = jnp.zeros((n_dst, z.shape[1]), z.dtype).at[dst].add(z[src])
    deg = jnp.zeros((n_dst,), z.dtype).at[dst].add(1.0)
    h_neigh = agg / jnp.clip(deg, 1.0, None)[:, None]
    return x_dst @ Wself + h_neigh + b


def _final_kernel(xw_ref, xd_ref, w_ref, b_ref, ow_ref, od_ref):
    w = w_ref[...]
    b = b_ref[0]
    ow_ref[...] = jax.nn.sigmoid(xw_ref[...] @ w + b)
    od_ref[...] = jax.nn.sigmoid(xd_ref[...] @ w + b)


def kernel(x_word, x_doc, edges_w2w, edges_w2d, edges_w2wr, edges_d2w,
           Wself1, Wneigh1, b1, Wself2, Wneigh2, b2, lin_w, lin_b):
    xw, xd = x_word, x_doc
    for Ws, Wn, b in ((Wself1, Wneigh1, b1), (Wself2, Wneigh2, b2)):
        ow = (_sage(xw, xw, edges_w2w, Ws[0], Wn[0], b[0], N_WORD)
              + _sage(xw, xw, edges_w2wr, Ws[2], Wn[2], b[2], N_WORD)
              + _sage(xd, xw, edges_d2w, Ws[3], Wn[3], b[3], N_WORD))
        od = _sage(xw, xd, edges_w2d, Ws[1], Wn[1], b[1], N_DOC)
        xw, xd = jax.nn.relu(ow), jax.nn.relu(od)

    out_word, out_doc = pl.pallas_call(
        _final_kernel,
        out_shape=(jax.ShapeDtypeStruct((N_WORD, 1), jnp.float32),
                   jax.ShapeDtypeStruct((N_DOC, 1), jnp.float32)),
    )(xw, xd, lin_w, lin_b)
    return out_word, out_doc


# speedup vs baseline: 2.0200x; 1.8460x over previous
import jax
import jax.numpy as jnp
from jax import lax
from jax.experimental import pallas as pl
from jax.experimental.pallas import tpu as pltpu
from jax.experimental.pallas import tpu_sc as plsc

N_NODE = 10000
NPAD = 10240
E = 160000
NW = 32
NS = 16
EPW = 5120
EPAD = NW * EPW
CH = 128
NCH = EPW // CH
RS = NPAD // NS
F = 128
DEGL = 4 * NPAD
DPS = DEGL // NS
M_BLK = 1024


def _mm_multi(x, wstack):
    R, K = wstack.shape[0], wstack.shape[1]

    def body(x_ref, w_ref, *outs):
        xv = x_ref[...]
        for i, o in enumerate(outs):
            o[...] = jnp.dot(xv, w_ref[i], preferred_element_type=jnp.float32)

    return pl.pallas_call(
        body,
        grid=(NPAD // M_BLK,),
        in_specs=[pl.BlockSpec((M_BLK, K), lambda m: (m, 0)),
                  pl.BlockSpec((R, K, F), lambda m: (0, 0, 0))],
        out_specs=tuple(pl.BlockSpec((M_BLK, F), lambda m: (m, 0))
                        for _ in range(R)),
        out_shape=tuple(jax.ShapeDtypeStruct((NPAD, F), jnp.float32)
                        for _ in range(R)),
    )(x, wstack)


def _agg_body(with_deg, *refs):
    if with_deg:
        (z0, z1, z2, z3, src_h, dst_h, dd_h, zer_h, zerd_h, acc_out, deg_out,
         src_v, dst_v, dd_v, rows_v, ones_v, acc_sh, deg_sh) = refs
    else:
        (z0, z1, z2, z3, src_h, dst_h, zer_h, acc_out,
         src_v, dst_v, rows_v, acc_sh) = refs
    c = lax.axis_index("c")
    s = lax.axis_index("s")
    wid = c * NS + s
    tables = (z0, z1, z2, z3)

    if with_deg:
        one16 = jnp.full((16,), 1.0, jnp.float32)
        for k in range(CH // 16):
            ones_v[pl.ds(k * 16, 16)] = one16
        pltpu.sync_copy(zerd_h, deg_sh.at[pl.ds(s * DPS, DPS)])

    for r in range(4):
        pltpu.sync_copy(zer_h, acc_sh.at[pl.ds(s * RS, RS)])
        pltpu.sync_copy(src_h.at[r, wid], src_v)
        pltpu.sync_copy(dst_h.at[r, wid], dst_v)
        if with_deg:
            pltpu.sync_copy(dd_h.at[r, wid], dd_v)
        plsc.subcore_barrier()

        zt = tables[r]

        def step(j, carry):
            pltpu.sync_copy(zt.at[src_v.at[j]], rows_v)
            pltpu.sync_copy(rows_v, acc_sh.at[dst_v.at[j]], add=True)
            if with_deg:
                pltpu.sync_copy(ones_v, deg_sh.at[dd_v.at[j]], add=True)
            return carry
        lax.fori_loop(0, NCH, step, 0)

        plsc.subcore_barrier()
        pltpu.sync_copy(acc_sh.at[pl.ds(s * RS, RS)],
                        acc_out.at[c, r, pl.ds(s * RS, RS)])
        plsc.subcore_barrier()

    if with_deg:
        pltpu.sync_copy(deg_sh.at[pl.ds(s * DPS, DPS)],
                        deg_out.at[c, pl.ds(s * DPS, DPS)])


def _agg_call(z0, z1, z2, z3, srcs, dsts, dstdeg, zeros_hbm, zeros_deg,
              with_deg):
    mesh = plsc.VectorSubcoreMesh(core_axis_name="c", subcore_axis_name="s")
    acc_t = jax.ShapeDtypeStruct((2, 4, NPAD, F), jnp.float32)
    deg_t = jax.ShapeDtypeStruct((2, DEGL), jnp.float32)
    scratch = [
        pltpu.VMEM((NCH, CH), jnp.int32),
        pltpu.VMEM((NCH, CH), jnp.int32),
    ]
    if with_deg:
        scratch.append(pltpu.VMEM((NCH, CH), jnp.int32))
    scratch.append(pltpu.VMEM((CH, F), jnp.float32))
    if with_deg:
        scratch.append(pltpu.VMEM((CH,), jnp.float32))
    scratch.append(pltpu.VMEM_SHARED((NPAD, F), jnp.float32))
    if with_deg:
        scratch.append(pltpu.VMEM_SHARED((DEGL,), jnp.float32))

    def body(*refs):
        _agg_body(with_deg, *refs)

    kern = pl.kernel(body,
                     out_type=(acc_t, deg_t) if with_deg else acc_t,
                     mesh=mesh, scratch_types=scratch)
    if with_deg:
        return kern(z0, z1, z2, z3, srcs, dsts, dstdeg, zeros_hbm, zeros_deg)
    return kern(z0, z1, z2, z3, srcs, dsts, zeros_hbm)


def _combine1(sw, sd, acc, deg4, bw, bd):
    def body(sw_ref, sd_ref, acc_ref, deg_ref, bw_ref, bd_ref, ow_ref, od_ref):
        acc_v = acc_ref[...]
        deg_v = deg_ref[...]
        sv = sw_ref[...] + bw_ref[...]
        for r in (0, 2, 3):
            d = jnp.clip(deg_v[0, r] + deg_v[1, r], 1.0, None)
            sv = sv + (acc_v[0, r] + acc_v[1, r]) / d
        ow_ref[...] = jax.nn.relu(sv)
        d1 = jnp.clip(deg_v[0, 1] + deg_v[1, 1], 1.0, None)
        od_ref[...] = jax.nn.relu(
            sd_ref[...] + bd_ref[...] + (acc_v[0, 1] + acc_v[1, 1]) / d1)

    return pl.pallas_call(
        body,
        grid=(NPAD // M_BLK,),
        in_specs=[pl.BlockSpec((M_BLK, F), lambda m: (m, 0)),
                  pl.BlockSpec((M_BLK, F), lambda m: (m, 0)),
                  pl.BlockSpec((2, 4, M_BLK, F), lambda m: (0, 0, m, 0)),
                  pl.BlockSpec((2, 4, M_BLK, 1), lambda m: (0, 0, m, 0)),
                  pl.BlockSpec((1, F), lambda m: (0, 0)),
                  pl.BlockSpec((1, F), lambda m: (0, 0))],
        out_specs=tuple(pl.BlockSpec((M_BLK, F), lambda m: (m, 0))
                        for _ in range(2)),
        out_shape=tuple(jax.ShapeDtypeStruct((NPAD, F), jnp.float32)
                        for _ in range(2)),
    )(sw, sd, acc, deg4, bw, bd)


def _combine2(sw, sd, acc, deg4, bw, bd, lw, lb):
    def body(sw_ref, sd_ref, acc_ref, deg_ref, bw_ref, bd_ref, lw_ref, lb_ref,
             ow_ref, od_ref):
        acc_v = acc_ref[...]
        deg_v = deg_ref[...]
        sv = sw_ref[...] + bw_ref[...]
        for r in (0, 2, 3):
            d = jnp.clip(deg_v[0, r] + deg_v[1, r], 1.0, None)
            sv = sv + (acc_v[0, r] + acc_v[1, r]) / d
        hw = jax.nn.relu(sv)
        d1 = jnp.clip(deg_v[0, 1] + deg_v[1, 1], 1.0, None)
        hd = jax.nn.relu(
            sd_ref[...] + bd_ref[...] + (acc_v[0, 1] + acc_v[1, 1]) / d1)
        lwv = lw_ref[...]
        lbv = lb_ref[0, 0]
        ow_ref[...] = jax.nn.sigmoid(
            jnp.dot(hw, lwv, preferred_element_type=jnp.float32) + lbv)
        od_ref[...] = jax.nn.sigmoid(
            jnp.dot(hd, lwv, preferred_element_type=jnp.float32) + lbv)

    return pl.pallas_call(
        body,
        grid=(NPAD // M_BLK,),
        in_specs=[pl.BlockSpec((M_BLK, F), lambda m: (m, 0)),
                  pl.BlockSpec((M_BLK, F), lambda m: (m, 0)),
                  pl.BlockSpec((2, 4, M_BLK, F), lambda m: (0, 0, m, 0)),
                  pl.BlockSpec((2, 4, M_BLK, 1), lambda m: (0, 0, m, 0)),
                  pl.BlockSpec((1, F), lambda m: (0, 0)),
                  pl.BlockSpec((1, F), lambda m: (0, 0)),
                  pl.BlockSpec((F, 1), lambda m: (0, 0)),
                  pl.BlockSpec((1, 1), lambda m: (0, 0))],
        out_specs=tuple(pl.BlockSpec((M_BLK, 1), lambda m: (m, 0))
                        for _ in range(2)),
        out_shape=tuple(jax.ShapeDtypeStruct((NPAD, 1), jnp.float32)
                        for _ in range(2)),
    )(sw, sd, acc, deg4, bw, bd, lw, lb)


def kernel(x_word, x_doc, edges_w2w, edges_w2d, edges_w2wr, edges_d2w,
           Wself1, Wneigh1, b1, Wself2, Wneigh2, b2, lin_w, lin_b):
    f32 = jnp.float32
    xw = jnp.zeros((NPAD, 256), f32).at[:N_NODE].set(x_word)
    xd = jnp.zeros((NPAD, 256), f32).at[:N_NODE].set(x_doc)

    def prep(e):
        src = jnp.concatenate(
            [e[0], jnp.zeros((EPAD - E,), jnp.int32)]).reshape(NW, NCH, CH)
        dst = jnp.concatenate(
            [e[1], jnp.full((EPAD - E,), N_NODE, jnp.int32)]).reshape(NW, NCH, CH)
        return src, dst

    prepped = [prep(e) for e in (edges_w2w, edges_w2d, edges_w2wr, edges_d2w)]
    srcs = jnp.stack([p[0] for p in prepped])
    dsts = jnp.stack([p[1] for p in prepped])
    dstdeg = dsts + (jnp.arange(4, dtype=jnp.int32) * NPAD)[:, None, None, None]
    zeros_hbm = jnp.zeros((RS, F), f32)
    zeros_deg = jnp.zeros((DPS,), f32)

    ww1 = jnp.stack([Wneigh1[0], Wneigh1[1], Wneigh1[2],
                     Wself1[0] + Wself1[2] + Wself1[3]])
    wd1 = jnp.stack([Wneigh1[3], Wself1[1]])
    bw1 = (b1[0] + b1[2] + b1[3]).reshape(1, F)
    bd1 = b1[1].reshape(1, F)
    ww2 = jnp.stack([Wneigh2[0], Wneigh2[1], Wneigh2[2],
                     Wself2[0] + Wself2[2] + Wself2[3]])
    wd2 = jnp.stack([Wneigh2[3], Wself2[1]])
    bw2 = (b2[0] + b2[2] + b2[3]).reshape(1, F)
    bd2 = b2[1].reshape(1, F)

    z0, z1, z2, sw = _mm_multi(xw, ww1)
    z3, sd = _mm_multi(xd, wd1)
    acc, deg = _agg_call(z0, z1, z2, z3, srcs, dsts, dstdeg, zeros_hbm,
                         zeros_deg, with_deg=True)
    deg4 = deg.reshape(2, 4, NPAD, 1)
    xw2, xd2 = _combine1(sw, sd, acc, deg4, bw1, bd1)

    z0, z1, z2, sw = _mm_multi(xw2, ww2)
    z3, sd = _mm_multi(xd2, wd2)
    acc2 = _agg_call(z0, z1, z2, z3, srcs, dsts, None, zeros_hbm, None,
                     with_deg=False)
    ow, od = _combine2(sw, sd, acc2, deg4, bw2, bd2,
                       lin_w, lin_b.reshape(1, 1))
    return ow[:N_NODE], od[:N_NODE]
